# Initial kernel scaffold; baseline (speedup 1.0000x reference)
import jax
import jax.numpy as jnp
from jax.experimental import pallas as pl


def kernel(x, W1, b1, W2, b2, Wq, bq, mem_keys, mem_vals, importance, Wout, bout):
    raise NotImplementedError("write your pallas kernel here")


# speedup vs baseline: 2.1481x; 2.1481x over previous
import jax
import jax.numpy as jnp
from jax import lax
from jax.experimental import pallas as pl
from jax.experimental.pallas import tpu as pltpu

IN_SIZE = 1024
HID = 1024
MEM_SIZE = 65536
MEM_DIM = 512
OUT_SIZE = 1024
TOP_K = 3
BLK = 4096
NBLK = MEM_SIZE // BLK
NEG_INF = float("-inf")
IMAX = 2**31 - 1


def _body(xlast_ref, W1_ref, b1_ref, W2_ref, b2_ref, Wq_ref, bq_ref,
          keys_ref, imp_ref, mem_vals_ref, Wout_ref, bout_ref,
          out_ref,
          qn_s, h2_s, tv_s, ti_s, row_s, sem):
    step = pl.program_id(0)

    @pl.when(step == 0)
    def _init():
        x = xlast_ref[...]
        h1 = jnp.maximum(
            jnp.dot(x, W1_ref[...], preferred_element_type=jnp.float32)
            + b1_ref[...], 0.0)
        h2 = jnp.dot(h1, W2_ref[...], preferred_element_type=jnp.float32) \
            + b2_ref[...]
        h2_s[...] = h2
        q = jnp.dot(h2, Wq_ref[...], preferred_element_type=jnp.float32) \
            + bq_ref[...]
        qnorm = jnp.sqrt(jnp.sum(q * q))
        qn_s[...] = q / jnp.maximum(qnorm, 1e-12)
        for j in range(TOP_K):
            tv_s[j] = NEG_INF
            ti_s[j] = 0

    blk = keys_ref[...]
    qn = qn_s[...]
    dn = (((1,), (1,)), ((), ()))
    dots = lax.dot_general(qn, blk, dn,
                           preferred_element_type=jnp.float32)
    sq = blk * blk
    ones = jnp.ones((1, MEM_DIM), dtype=jnp.float32)
    rn = lax.dot_general(ones, sq, dn,
                         preferred_element_type=jnp.float32)
    w = dots / jnp.maximum(jnp.sqrt(rn), 1e-12) * imp_ref[0]
    gidx = step * BLK + lax.broadcasted_iota(jnp.int32, (1, BLK), 1)

    for _ in range(TOP_K):
        m = jnp.max(w)
        sel = jnp.min(jnp.where(w == m, gidx, IMAX))
        v1, v2, v3 = tv_s[0], tv_s[1], tv_s[2]
        i1, i2, i3 = ti_s[0], ti_s[1], ti_s[2]
        c1 = m > v1
        c2 = m > v2
        c3 = m > v3
        tv_s[2] = jnp.where(c2, v2, jnp.where(c3, m, v3))
        ti_s[2] = jnp.where(c2, i2, jnp.where(c3, sel, i3))
        tv_s[1] = jnp.where(c1, v1, jnp.where(c2, m, v2))
        ti_s[1] = jnp.where(c1, i1, jnp.where(c2, sel, i2))
        tv_s[0] = jnp.where(c1, m, v1)
        ti_s[0] = jnp.where(c1, sel, i1)
        w = jnp.where(gidx == sel, NEG_INF, w)

    @pl.when(step == NBLK - 1)
    def _final():
        cps = [pltpu.make_async_copy(
            mem_vals_ref.at[pl.ds(ti_s[j], 1)], row_s.at[pl.ds(j, 1)], sem)
            for j in range(TOP_K)]
        for cp in cps:
            cp.start()
        for cp in cps:
            cp.wait()
        m0 = tv_s[0]
        e = [jnp.exp(jnp.full((1, MEM_DIM), tv_s[j] - m0, dtype=jnp.float32))
             for j in range(TOP_K)]
        den = e[0] + e[1] + e[2]
        retrieved = (e[0] * row_s[0:1, :] + e[1] * row_s[1:2, :]
                     + e[2] * row_s[2:3, :]) / den
        out = (jnp.dot(h2_s[...], Wout_ref[0:HID, :],
                       preferred_element_type=jnp.float32)
               + jnp.dot(retrieved, Wout_ref[HID:HID + MEM_DIM, :],
                         preferred_element_type=jnp.float32)
               + bout_ref[...])
        out_ref[...] = out


def kernel(x, W1, b1, W2, b2, Wq, bq, mem_keys, mem_vals, importance, Wout, bout):
    x_last = x[:, -1, :]
    imp3 = importance.reshape(NBLK, 1, BLK)
    full = lambda i: (0, 0)
    grid_spec = pltpu.PrefetchScalarGridSpec(
        num_scalar_prefetch=0,
        grid=(NBLK,),
        in_specs=[
            pl.BlockSpec((1, IN_SIZE), full),
            pl.BlockSpec((IN_SIZE, HID), full),
            pl.BlockSpec((1, HID), full),
            pl.BlockSpec((HID, HID), full),
            pl.BlockSpec((1, HID), full),
            pl.BlockSpec((HID, MEM_DIM), full),
            pl.BlockSpec((1, MEM_DIM), full),
            pl.BlockSpec((BLK, MEM_DIM), lambda i: (i, 0)),
            pl.BlockSpec((1, 1, BLK), lambda i: (i, 0, 0)),
            pl.BlockSpec(memory_space=pl.ANY),
            pl.BlockSpec((HID + MEM_DIM, OUT_SIZE), full),
            pl.BlockSpec((1, OUT_SIZE), full),
        ],
        out_specs=pl.BlockSpec((1, OUT_SIZE), full),
        scratch_shapes=[
            pltpu.VMEM((1, MEM_DIM), jnp.float32),
            pltpu.VMEM((1, HID), jnp.float32),
            pltpu.SMEM((TOP_K,), jnp.float32),
            pltpu.SMEM((TOP_K,), jnp.int32),
            pltpu.VMEM((8, MEM_DIM), jnp.float32),
            pltpu.SemaphoreType.DMA,
        ],
    )
    return pl.pallas_call(
        _body,
        grid_spec=grid_spec,
        out_shape=jax.ShapeDtypeStruct((1, OUT_SIZE), jnp.float32),
        compiler_params=pltpu.CompilerParams(
            dimension_semantics=("arbitrary",),
        ),
    )(x_last, W1, b1.reshape(1, HID), W2, b2.reshape(1, HID),
      Wq, bq.reshape(1, MEM_DIM), mem_keys, imp3, mem_vals,
      Wout, bout.reshape(1, OUT_SIZE))
